# Initial kernel scaffold; baseline (speedup 1.0000x reference)
import jax
import jax.numpy as jnp
from jax.experimental import pallas as pl


def kernel(x, edge_index, W1, b1, W2, b2, Wl, bl):
    raise NotImplementedError("write your pallas kernel here")


# speedup vs baseline: 231.3378x; 231.3378x over previous
import jax
import jax.numpy as jnp
from jax import lax
from jax.experimental import pallas as pl
from jax.experimental.pallas import tpu as pltpu
from jax.experimental.pallas import tpu_sc as plsc

N_NODES = 100000
N_EDGES = 6400000
NC = 2
NS = 16
N_SL = 6256
N_PAD = N_SL * NS
RW = 8
NPR = 128 // RW
N_ROWS = N_PAD // NPR
N_FULL = N_NODES // NPR
E_CORE = N_EDGES // NC
E_TILE = E_CORE // NS
CH = 8000
N_CHUNKS = E_TILE // CH
CHM = 2000
N_CHUNKS_M = E_TILE // CHM
N_ST = N_SL // 8

_mesh = plsc.VectorSubcoreMesh(core_axis_name="c", subcore_axis_name="s")
_sc_params = pltpu.CompilerParams(use_tc_tiling_on_sc=False)


def _zero_vmem(buf, n):
    def fill(i, carry):
        buf[pl.ds(i * 16, 16)] = jnp.zeros((16,), jnp.float32)
        return carry

    lax.fori_loop(0, n // 16, fill, 0)


def _deg_body(dst_h, degp_h, deg_sh, idx_v, ones_v, zb1):
    c = lax.axis_index("c")
    s = lax.axis_index("s")
    r0 = pl.multiple_of(s * N_SL, 8)
    _zero_vmem(zb1, N_SL)
    pltpu.sync_copy(zb1, deg_sh.at[pl.ds(r0, N_SL)])

    def fill(i, carry):
        ones_v[pl.ds(i * 16, 16)] = jnp.ones((16,), jnp.float32)
        return carry

    lax.fori_loop(0, CH // 16, fill, 0)
    plsc.subcore_barrier()
    base0 = c * E_CORE + s * E_TILE

    def chunk(k, carry):
        b = pl.multiple_of(base0 + k * CH, 8)
        pltpu.sync_copy(dst_h.at[pl.ds(b, CH)], idx_v)
        pltpu.sync_copy(ones_v, deg_sh.at[idx_v], add=True)
        return carry

    lax.fori_loop(0, N_CHUNKS, chunk, 0)
    plsc.subcore_barrier()
    o0 = pl.multiple_of(c * N_PAD + r0, 8)
    pltpu.sync_copy(deg_sh.at[pl.ds(r0, N_SL)], zb1)
    pltpu.sync_copy(zb1, degp_h.at[pl.ds(o0, N_SL)])


_deg_call = pl.kernel(
    _deg_body,
    out_type=pltpu.MemorySpace.HBM((NC * N_PAD,), jnp.float32),
    mesh=_mesh,
    scratch_types=[
        pltpu.VMEM_SHARED((N_PAD,), jnp.float32),
        pltpu.VMEM((CH,), jnp.int32),
        pltpu.VMEM((CH,), jnp.float32),
        pltpu.VMEM((N_SL,), jnp.float32),
    ],
    compiler_params=_sc_params,
)


def _msg_body(src_h, dst_h, z_h, zero4_h, Ap_h, z_sh, A_sh, sidx, didx,
              zrows, zb4):
    c = lax.axis_index("c")
    s = lax.axis_index("s")
    r0 = pl.multiple_of(s * N_SL, 8)
    for j in range(8):
        rj = r0 + j * N_ST
        pltpu.sync_copy(z_h.at[pl.ds(rj, N_ST), :], zb4)
        pltpu.sync_copy(zb4, z_sh.at[pl.ds(rj, N_ST), :])
        pltpu.sync_copy(zero4_h.at[pl.ds(rj, N_ST), :], zb4)
        pltpu.sync_copy(zb4, A_sh.at[pl.ds(rj, N_ST), :])
    plsc.subcore_barrier()
    base0 = c * E_CORE + s * E_TILE

    def chunk(k, carry):
        b = pl.multiple_of(base0 + k * CHM, 8)
        pltpu.sync_copy(src_h.at[pl.ds(b, CHM)], sidx)
        pltpu.sync_copy(dst_h.at[pl.ds(b, CHM)], didx)
        pltpu.sync_copy(z_sh.at[sidx], zrows)
        pltpu.sync_copy(zrows, A_sh.at[didx], add=True)
        return carry

    lax.fori_loop(0, N_CHUNKS_M, chunk, 0)
    plsc.subcore_barrier()
    for j in range(8):
        rj = r0 + j * N_ST
        pltpu.sync_copy(A_sh.at[pl.ds(rj, N_ST), :], zb4)
        pltpu.sync_copy(zb4, Ap_h.at[c, pl.ds(rj, N_ST), :])


_msg_call = pl.kernel(
    _msg_body,
    out_type=pltpu.MemorySpace.HBM((NC, N_PAD, RW), jnp.float32),
    mesh=_mesh,
    scratch_types=[
        pltpu.VMEM_SHARED((N_PAD, RW), jnp.float32),
        pltpu.VMEM_SHARED((N_PAD, RW), jnp.float32),
        pltpu.VMEM((CHM,), jnp.int32),
        pltpu.VMEM((CHM,), jnp.int32),
        pltpu.VMEM((CHM, RW), jnp.float32),
        pltpu.VMEM((N_ST, RW), jnp.float32),
    ],
    compiler_params=_sc_params,
)


def _t_body(src_h, dst_h, dinv_h, tp_h, dinv_sh, t_sh, sidx, didx, dbuf, zb1):
    c = lax.axis_index("c")
    s = lax.axis_index("s")
    r0 = pl.multiple_of(s * N_SL, 8)
    pltpu.sync_copy(dinv_h.at[pl.ds(r0, N_SL)], zb1)
    pltpu.sync_copy(zb1, dinv_sh.at[pl.ds(r0, N_SL)])
    _zero_vmem(zb1, N_SL)
    pltpu.sync_copy(zb1, t_sh.at[pl.ds(r0, N_SL)])
    plsc.subcore_barrier()
    base0 = c * E_CORE + s * E_TILE

    def chunk(k, carry):
        b = pl.multiple_of(base0 + k * CH, 8)
        pltpu.sync_copy(src_h.at[pl.ds(b, CH)], sidx)
        pltpu.sync_copy(dst_h.at[pl.ds(b, CH)], didx)
        pltpu.sync_copy(dinv_sh.at[didx], dbuf)
        pltpu.sync_copy(dbuf, t_sh.at[sidx], add=True)
        return carry

    lax.fori_loop(0, N_CHUNKS, chunk, 0)
    plsc.subcore_barrier()
    o0 = pl.multiple_of(c * N_PAD + r0, 8)
    pltpu.sync_copy(t_sh.at[pl.ds(r0, N_SL)], zb1)
    pltpu.sync_copy(zb1, tp_h.at[pl.ds(o0, N_SL)])


_t_call = pl.kernel(
    _t_body,
    out_type=pltpu.MemorySpace.HBM((NC * N_PAD,), jnp.float32),
    mesh=_mesh,
    scratch_types=[
        pltpu.VMEM_SHARED((N_PAD,), jnp.float32),
        pltpu.VMEM_SHARED((N_PAD,), jnp.float32),
        pltpu.VMEM((CH,), jnp.int32),
        pltpu.VMEM((CH,), jnp.int32),
        pltpu.VMEM((CH,), jnp.float32),
        pltpu.VMEM((N_SL,), jnp.float32),
    ],
    compiler_params=_sc_params,
)


_HI = lax.Precision.HIGHEST


def _prep_body(degp_ref, xv_ref, e8_ref, dinv_ref, z_ref):
    dp = degp_ref[...]
    deg = dp[0] + dp[1] + 1.0
    dinvR = lax.rsqrt(deg)
    dinv_ref[...] = dinvR
    dinv128 = jnp.dot(dinvR, e8_ref[...], precision=_HI,
                      preferred_element_type=jnp.float32)
    z_ref[...] = xv_ref[...] * dinv128


def _finish_body(Ap_ref, zv_ref, dinvR_ref, tp_ref, e8_ref, e16_ref,
                 fold_ref, wbig_ref, b1big_ref, W2_ref, b2_ref, Wl_ref,
                 bl_ref, out_ref):
    dinvR = dinvR_ref[...]
    S = Ap_ref[0] + Ap_ref[1] + zv_ref[...]
    Ssc = S * jnp.dot(dinvR, e8_ref[...], precision=_HI,
                      preferred_element_type=jnp.float32)
    H = jnp.maximum(
        jnp.dot(Ssc, wbig_ref[...], precision=_HI,
                preferred_element_type=jnp.float32)
        + b1big_ref[...][None, :], 0.0)
    tR = tp_ref[0] + tp_ref[1]
    dinvA = jnp.dot(dinvR, e16_ref[...], precision=_HI,
                    preferred_element_type=jnp.float32)
    tA = jnp.dot(tR, e16_ref[...], precision=_HI,
                 preferred_element_type=jnp.float32)
    wA = dinvA * (dinvA + tA)
    rowmask = (lax.broadcasted_iota(jnp.int32, (N_ROWS, 1), 0) < N_FULL)
    P = jnp.where(rowmask, H * wA, 0.0)
    c512 = jnp.sum(P, axis=0)
    c = jnp.dot(c512[None, :], fold_ref[...], precision=_HI,
                preferred_element_type=jnp.float32)
    G = jnp.dot(c, W2_ref[...].T, precision=_HI,
                preferred_element_type=jnp.float32) \
        + float(N_NODES) * b2_ref[...][None, :]
    L = jnp.dot(G, Wl_ref[...].T, precision=_HI,
                preferred_element_type=jnp.float32) \
        + bl_ref[...][None, :]
    m = jnp.max(L, axis=1, keepdims=True)
    out_ref[...] = L - m - jnp.log(jnp.sum(jnp.exp(L - m), axis=1,
                                           keepdims=True))


def kernel(x, edge_index, W1, b1, W2, b2, Wl, bl):
    f32 = jnp.float32
    src = edge_index[0].astype(jnp.int32)
    dst = edge_index[1].astype(jnp.int32)
    xv = jnp.pad(x.astype(f32), ((0, N_PAD - N_NODES), (0, RW - 3))) \
        .reshape(N_ROWS, 128)
    eyeN = jnp.eye(NPR, dtype=f32)
    e8 = jnp.kron(eyeN, jnp.ones((1, RW), f32))
    e16 = jnp.kron(eyeN, jnp.ones((1, 16), f32))
    fold = jnp.tile(jnp.eye(16, dtype=f32), (NPR, 1))
    W1p = jnp.pad(W1.astype(f32), ((0, 0), (0, RW - 3)))
    wbig = jnp.kron(eyeN, W1p.T)
    b1big = jnp.tile(b1.astype(f32), NPR)
    zeros4 = jnp.zeros((N_PAD, RW), f32)

    degp = _deg_call(dst).reshape(NC, N_ROWS, NPR)

    dinvR, zv = pl.pallas_call(
        _prep_body,
        out_shape=(
            jax.ShapeDtypeStruct((N_ROWS, NPR), f32),
            jax.ShapeDtypeStruct((N_ROWS, 128), f32),
        ),
    )(degp, xv, e8)

    hbm = pltpu.MemorySpace.HBM
    z_rows = pltpu.with_memory_space_constraint(zv.reshape(N_PAD, RW), hbm)
    dinv_flat = pltpu.with_memory_space_constraint(dinvR.reshape(N_PAD), hbm)
    zeros4 = pltpu.with_memory_space_constraint(zeros4, hbm)

    Ap = _msg_call(src, dst, z_rows, zeros4)
    tp = _t_call(src, dst, dinv_flat)

    Ap_v = Ap.reshape(NC, N_ROWS, 128)
    tp_v = tp.reshape(NC, N_ROWS, NPR)

    out = pl.pallas_call(
        _finish_body,
        out_shape=jax.ShapeDtypeStruct((1, 7), f32),
    )(Ap_v, zv, dinvR, tp_v, e8, e16, fold, wbig, b1big, W2, b2, Wl, bl)
    return out


# speedup vs baseline: 269.7336x; 1.1660x over previous
import jax
import jax.numpy as jnp
from jax import lax
from jax.experimental import pallas as pl
from jax.experimental.pallas import tpu as pltpu
from jax.experimental.pallas import tpu_sc as plsc

N_NODES = 100000
N_EDGES = 6400000
NC = 2
NS = 16
N_SL = 6256
N_PAD = N_SL * NS
RW = 8
NPR = 128 // RW
N_ROWS = N_PAD // NPR
N_FULL = N_NODES // NPR
E_CORE = N_EDGES // NC
E_TILE = E_CORE // NS
CH = 8000
N_CHUNKS = E_TILE // CH
CHM = 1000
N_CHUNKS_M = E_TILE // CHM
N_ST = N_SL // 8

_mesh = plsc.VectorSubcoreMesh(core_axis_name="c", subcore_axis_name="s")
_sc_params = pltpu.CompilerParams(use_tc_tiling_on_sc=False)


def _zero_vmem(buf, n):
    def fill(i, carry):
        buf[pl.ds(i * 16, 16)] = jnp.zeros((16,), jnp.float32)
        return carry

    lax.fori_loop(0, n // 16, fill, 0)


def _deg_body(dst_h, degp_h, deg_sh, idx_v, ones_v, zb1, ld_sem, sc_sem):
    c = lax.axis_index("c")
    s = lax.axis_index("s")
    r0 = pl.multiple_of(s * N_SL, 8)
    _zero_vmem(zb1, N_SL)
    pltpu.sync_copy(zb1, deg_sh.at[pl.ds(r0, N_SL)])

    def fill(i, carry):
        ones_v[pl.ds(i * 16, 16)] = jnp.ones((16,), jnp.float32)
        return carry

    lax.fori_loop(0, CH // 16, fill, 0)
    plsc.subcore_barrier()
    base0 = c * E_CORE + s * E_TILE
    b0 = pl.multiple_of(base0, 8)
    pltpu.async_copy(dst_h.at[pl.ds(b0, CH)], idx_v.at[0], ld_sem.at[0])

    def chunk(k, carry):
        p = lax.rem(k, 2)
        q = 1 - p
        b = pl.multiple_of(base0 + k * CH, 8)
        pltpu.make_async_copy(dst_h.at[pl.ds(b, CH)], idx_v.at[p],
                              ld_sem.at[p]).wait()
        pltpu.async_copy(ones_v, deg_sh.at[idx_v.at[p]], sc_sem.at[p],
                         add=True)

        @pl.when(k + 1 < N_CHUNKS)
        def _prefetch():
            @pl.when(k >= 1)
            def _drain():
                pltpu.make_async_copy(ones_v, deg_sh.at[idx_v.at[q]],
                                      sc_sem.at[q]).wait()
            bn = pl.multiple_of(base0 + (k + 1) * CH, 8)
            pltpu.async_copy(dst_h.at[pl.ds(bn, CH)], idx_v.at[q],
                             ld_sem.at[q])

        return carry

    lax.fori_loop(0, N_CHUNKS, chunk, 0)
    pltpu.make_async_copy(ones_v, deg_sh.at[idx_v.at[(N_CHUNKS - 2) % 2]],
                          sc_sem.at[(N_CHUNKS - 2) % 2]).wait()
    pltpu.make_async_copy(ones_v, deg_sh.at[idx_v.at[(N_CHUNKS - 1) % 2]],
                          sc_sem.at[(N_CHUNKS - 1) % 2]).wait()
    plsc.subcore_barrier()
    o0 = pl.multiple_of(c * N_PAD + r0, 8)
    pltpu.sync_copy(deg_sh.at[pl.ds(r0, N_SL)], zb1)
    pltpu.sync_copy(zb1, degp_h.at[pl.ds(o0, N_SL)])


_deg_call = pl.kernel(
    _deg_body,
    out_type=pltpu.MemorySpace.HBM((NC * N_PAD,), jnp.float32),
    mesh=_mesh,
    scratch_types=[
        pltpu.VMEM_SHARED((N_PAD,), jnp.float32),
        pltpu.VMEM((2, CH), jnp.int32),
        pltpu.VMEM((CH,), jnp.float32),
        pltpu.VMEM((N_SL,), jnp.float32),
        pltpu.SemaphoreType.DMA((2,)),
        pltpu.SemaphoreType.DMA((2,)),
    ],
    compiler_params=_sc_params,
)


def _msg_body(src_h, dst_h, z_h, zero4_h, Ap_h, z_sh, A_sh, sidx, didx,
              zrows, zb4, ls_sem, ld_sem, g_sem, sc_sem):
    c = lax.axis_index("c")
    s = lax.axis_index("s")
    r0 = pl.multiple_of(s * N_SL, 8)
    for j in range(8):
        rj = r0 + j * N_ST
        pltpu.sync_copy(z_h.at[pl.ds(rj, N_ST), :], zb4)
        pltpu.sync_copy(zb4, z_sh.at[pl.ds(rj, N_ST), :])
        pltpu.sync_copy(zero4_h.at[pl.ds(rj, N_ST), :], zb4)
        pltpu.sync_copy(zb4, A_sh.at[pl.ds(rj, N_ST), :])
    plsc.subcore_barrier()
    base0 = c * E_CORE + s * E_TILE
    b0 = pl.multiple_of(base0, 8)
    pltpu.async_copy(src_h.at[pl.ds(b0, CHM)], sidx.at[0], ls_sem.at[0])
    pltpu.async_copy(dst_h.at[pl.ds(b0, CHM)], didx.at[0], ld_sem.at[0])

    def chunk(k, carry):
        p = lax.rem(k, 2)
        q = 1 - p
        b = pl.multiple_of(base0 + k * CHM, 8)
        pltpu.make_async_copy(src_h.at[pl.ds(b, CHM)], sidx.at[p],
                              ls_sem.at[p]).wait()
        pltpu.make_async_copy(dst_h.at[pl.ds(b, CHM)], didx.at[p],
                              ld_sem.at[p]).wait()
        pltpu.async_copy(z_sh.at[sidx.at[p]], zrows.at[p], g_sem).wait()
        pltpu.async_copy(zrows.at[p], A_sh.at[didx.at[p]], sc_sem.at[p],
                         add=True)

        @pl.when(k + 1 < N_CHUNKS_M)
        def _prefetch():
            @pl.when(k >= 1)
            def _drain():
                pltpu.make_async_copy(zrows.at[q], A_sh.at[didx.at[q]],
                                      sc_sem.at[q]).wait()
            bn = pl.multiple_of(base0 + (k + 1) * CHM, 8)
            pltpu.async_copy(src_h.at[pl.ds(bn, CHM)], sidx.at[q],
                             ls_sem.at[q])
            pltpu.async_copy(dst_h.at[pl.ds(bn, CHM)], didx.at[q],
                             ld_sem.at[q])

        return carry

    lax.fori_loop(0, N_CHUNKS_M, chunk, 0)
    pltpu.make_async_copy(zrows.at[(N_CHUNKS_M - 2) % 2],
                          A_sh.at[didx.at[(N_CHUNKS_M - 2) % 2]],
                          sc_sem.at[(N_CHUNKS_M - 2) % 2]).wait()
    pltpu.make_async_copy(zrows.at[(N_CHUNKS_M - 1) % 2],
                          A_sh.at[didx.at[(N_CHUNKS_M - 1) % 2]],
                          sc_sem.at[(N_CHUNKS_M - 1) % 2]).wait()
    plsc.subcore_barrier()
    for j in range(8):
        rj = r0 + j * N_ST
        pltpu.sync_copy(A_sh.at[pl.ds(rj, N_ST), :], zb4)
        pltpu.sync_copy(zb4, Ap_h.at[c, pl.ds(rj, N_ST), :])


_msg_call = pl.kernel(
    _msg_body,
    out_type=pltpu.MemorySpace.HBM((NC, N_PAD, RW), jnp.float32),
    mesh=_mesh,
    scratch_types=[
        pltpu.VMEM_SHARED((N_PAD, RW), jnp.float32),
        pltpu.VMEM_SHARED((N_PAD, RW), jnp.float32),
        pltpu.VMEM((2, CHM), jnp.int32),
        pltpu.VMEM((2, CHM), jnp.int32),
        pltpu.VMEM((2, CHM, RW), jnp.float32),
        pltpu.VMEM((N_ST, RW), jnp.float32),
        pltpu.SemaphoreType.DMA((2,)),
        pltpu.SemaphoreType.DMA((2,)),
        pltpu.SemaphoreType.DMA,
        pltpu.SemaphoreType.DMA((2,)),
    ],
    compiler_params=_sc_params,
)


def _t_body(src_h, dst_h, dinv_h, tp_h, dinv_sh, t_sh, sidx, didx, dbuf,
            zb1, ls_sem, ld_sem, g_sem, sc_sem):
    c = lax.axis_index("c")
    s = lax.axis_index("s")
    r0 = pl.multiple_of(s * N_SL, 8)
    pltpu.sync_copy(dinv_h.at[pl.ds(r0, N_SL)], zb1)
    pltpu.sync_copy(zb1, dinv_sh.at[pl.ds(r0, N_SL)])
    _zero_vmem(zb1, N_SL)
    pltpu.sync_copy(zb1, t_sh.at[pl.ds(r0, N_SL)])
    plsc.subcore_barrier()
    base0 = c * E_CORE + s * E_TILE
    b0 = pl.multiple_of(base0, 8)
    pltpu.async_copy(src_h.at[pl.ds(b0, CH)], sidx.at[0], ls_sem.at[0])
    pltpu.async_copy(dst_h.at[pl.ds(b0, CH)], didx.at[0], ld_sem.at[0])

    def chunk(k, carry):
        p = lax.rem(k, 2)
        q = 1 - p
        b = pl.multiple_of(base0 + k * CH, 8)
        pltpu.make_async_copy(src_h.at[pl.ds(b, CH)], sidx.at[p],
                              ls_sem.at[p]).wait()
        pltpu.make_async_copy(dst_h.at[pl.ds(b, CH)], didx.at[p],
                              ld_sem.at[p]).wait()
        pltpu.async_copy(dinv_sh.at[didx.at[p]], dbuf.at[p], g_sem).wait()
        pltpu.async_copy(dbuf.at[p], t_sh.at[sidx.at[p]], sc_sem.at[p],
                         add=True)

        @pl.when(k + 1 < N_CHUNKS)
        def _prefetch():
            @pl.when(k >= 1)
            def _drain():
                pltpu.make_async_copy(dbuf.at[q], t_sh.at[sidx.at[q]],
                                      sc_sem.at[q]).wait()
            bn = pl.multiple_of(base0 + (k + 1) * CH, 8)
            pltpu.async_copy(src_h.at[pl.ds(bn, CH)], sidx.at[q],
                             ls_sem.at[q])
            pltpu.async_copy(dst_h.at[pl.ds(bn, CH)], didx.at[q],
                             ld_sem.at[q])

        return carry

    lax.fori_loop(0, N_CHUNKS, chunk, 0)
    pltpu.make_async_copy(dbuf.at[(N_CHUNKS - 2) % 2],
                          t_sh.at[sidx.at[(N_CHUNKS - 2) % 2]],
                          sc_sem.at[(N_CHUNKS - 2) % 2]).wait()
    pltpu.make_async_copy(dbuf.at[(N_CHUNKS - 1) % 2],
                          t_sh.at[sidx.at[(N_CHUNKS - 1) % 2]],
                          sc_sem.at[(N_CHUNKS - 1) % 2]).wait()
    plsc.subcore_barrier()
    o0 = pl.multiple_of(c * N_PAD + r0, 8)
    pltpu.sync_copy(t_sh.at[pl.ds(r0, N_SL)], zb1)
    pltpu.sync_copy(zb1, tp_h.at[pl.ds(o0, N_SL)])


_t_call = pl.kernel(
    _t_body,
    out_type=pltpu.MemorySpace.HBM((NC * N_PAD,), jnp.float32),
    mesh=_mesh,
    scratch_types=[
        pltpu.VMEM_SHARED((N_PAD,), jnp.float32),
        pltpu.VMEM_SHARED((N_PAD,), jnp.float32),
        pltpu.VMEM((2, CH), jnp.int32),
        pltpu.VMEM((2, CH), jnp.int32),
        pltpu.VMEM((2, CH), jnp.float32),
        pltpu.VMEM((N_SL,), jnp.float32),
        pltpu.SemaphoreType.DMA((2,)),
        pltpu.SemaphoreType.DMA((2,)),
        pltpu.SemaphoreType.DMA,
        pltpu.SemaphoreType.DMA((2,)),
    ],
    compiler_params=_sc_params,
)


_HI = lax.Precision.HIGHEST


def _prep_body(degp_ref, xv_ref, e8_ref, dinv_ref, z_ref):
    dp = degp_ref[...]
    deg = dp[0] + dp[1] + 1.0
    dinvR = lax.rsqrt(deg)
    dinv_ref[...] = dinvR
    dinv128 = jnp.dot(dinvR, e8_ref[...], precision=_HI,
                      preferred_element_type=jnp.float32)
    z_ref[...] = xv_ref[...] * dinv128


def _finish_body(Ap_ref, zv_ref, dinvR_ref, tp_ref, e8_ref, e16_ref,
                 fold_ref, wbig_ref, b1big_ref, W2_ref, b2_ref, Wl_ref,
                 bl_ref, out_ref):
    dinvR = dinvR_ref[...]
    S = Ap_ref[0] + Ap_ref[1] + zv_ref[...]
    Ssc = S * jnp.dot(dinvR, e8_ref[...], precision=_HI,
                      preferred_element_type=jnp.float32)
    H = jnp.maximum(
        jnp.dot(Ssc, wbig_ref[...], precision=_HI,
                preferred_element_type=jnp.float32)
        + b1big_ref[...][None, :], 0.0)
    tR = tp_ref[0] + tp_ref[1]
    dinvA = jnp.dot(dinvR, e16_ref[...], precision=_HI,
                    preferred_element_type=jnp.float32)
    tA = jnp.dot(tR, e16_ref[...], precision=_HI,
                 preferred_element_type=jnp.float32)
    wA = dinvA * (dinvA + tA)
    rowmask = (lax.broadcasted_iota(jnp.int32, (N_ROWS, 1), 0) < N_FULL)
    P = jnp.where(rowmask, H * wA, 0.0)
    c512 = jnp.sum(P, axis=0)
    c = jnp.dot(c512[None, :], fold_ref[...], precision=_HI,
                preferred_element_type=jnp.float32)
    G = jnp.dot(c, W2_ref[...].T, precision=_HI,
                preferred_element_type=jnp.float32) \
        + float(N_NODES) * b2_ref[...][None, :]
    L = jnp.dot(G, Wl_ref[...].T, precision=_HI,
                preferred_element_type=jnp.float32) \
        + bl_ref[...][None, :]
    m = jnp.max(L, axis=1, keepdims=True)
    out_ref[...] = L - m - jnp.log(jnp.sum(jnp.exp(L - m), axis=1,
                                           keepdims=True))


def kernel(x, edge_index, W1, b1, W2, b2, Wl, bl):
    f32 = jnp.float32
    src = edge_index[0].astype(jnp.int32)
    dst = edge_index[1].astype(jnp.int32)
    xv = jnp.pad(x.astype(f32), ((0, N_PAD - N_NODES), (0, RW - 3))) \
        .reshape(N_ROWS, 128)
    eyeN = jnp.eye(NPR, dtype=f32)
    e8 = jnp.kron(eyeN, jnp.ones((1, RW), f32))
    e16 = jnp.kron(eyeN, jnp.ones((1, 16), f32))
    fold = jnp.tile(jnp.eye(16, dtype=f32), (NPR, 1))
    W1p = jnp.pad(W1.astype(f32), ((0, 0), (0, RW - 3)))
    wbig = jnp.kron(eyeN, W1p.T)
    b1big = jnp.tile(b1.astype(f32), NPR)
    zeros4 = jnp.zeros((N_PAD, RW), f32)

    degp = _deg_call(dst).reshape(NC, N_ROWS, NPR)

    dinvR, zv = pl.pallas_call(
        _prep_body,
        out_shape=(
            jax.ShapeDtypeStruct((N_ROWS, NPR), f32),
            jax.ShapeDtypeStruct((N_ROWS, 128), f32),
        ),
    )(degp, xv, e8)

    hbm = pltpu.MemorySpace.HBM
    z_rows = pltpu.with_memory_space_constraint(zv.reshape(N_PAD, RW), hbm)
    dinv_flat = pltpu.with_memory_space_constraint(dinvR.reshape(N_PAD), hbm)
    zeros4 = pltpu.with_memory_space_constraint(zeros4, hbm)

    Ap = _msg_call(src, dst, z_rows, zeros4)
    tp = _t_call(src, dst, dinv_flat)

    Ap_v = Ap.reshape(NC, N_ROWS, 128)
    tp_v = tp.reshape(NC, N_ROWS, NPR)

    out = pl.pallas_call(
        _finish_body,
        out_shape=jax.ShapeDtypeStruct((1, 7), f32),
    )(Ap_v, zv, dinvR, tp_v, e8, e16, fold, wbig, b1big, W2, b2, Wl, bl)
    return out


# speedup vs baseline: 270.1403x; 1.0015x over previous
import jax
import jax.numpy as jnp
from jax import lax
from jax.experimental import pallas as pl
from jax.experimental.pallas import tpu as pltpu
from jax.experimental.pallas import tpu_sc as plsc

N_NODES = 100000
N_EDGES = 6400000
NC = 2
NS = 16
N_SL = 6256
N_PAD = N_SL * NS
RW = 8
NPR = 128 // RW
N_ROWS = N_PAD // NPR
N_FULL = N_NODES // NPR
E_CORE = N_EDGES // NC
E_TILE = E_CORE // NS
CH = 10000
N_CHUNKS = E_TILE // CH
CHM = 1000
N_CHUNKS_M = E_TILE // CHM
N_ST = N_SL // 8

_mesh = plsc.VectorSubcoreMesh(core_axis_name="c", subcore_axis_name="s")
_sc_params = pltpu.CompilerParams(use_tc_tiling_on_sc=False)


def _zero_vmem(buf, n):
    def fill(i, carry):
        buf[pl.ds(i * 16, 16)] = jnp.zeros((16,), jnp.float32)
        return carry

    lax.fori_loop(0, n // 16, fill, 0)


def _deg_body(dst_h, degp_h, deg_sh, idx_v, ones_v, zb1, ld_sem, sc_sem):
    c = lax.axis_index("c")
    s = lax.axis_index("s")
    r0 = pl.multiple_of(s * N_SL, 8)
    _zero_vmem(zb1, N_SL)
    pltpu.sync_copy(zb1, deg_sh.at[pl.ds(r0, N_SL)])

    def fill(i, carry):
        ones_v[pl.ds(i * 16, 16)] = jnp.ones((16,), jnp.float32)
        return carry

    lax.fori_loop(0, CH // 16, fill, 0)
    plsc.subcore_barrier()
    base0 = c * E_CORE + s * E_TILE
    b0 = pl.multiple_of(base0, 8)
    pltpu.async_copy(dst_h.at[pl.ds(b0, CH)], idx_v.at[0], ld_sem.at[0])

    def chunk(k, carry):
        p = lax.rem(k, 2)
        q = 1 - p
        b = pl.multiple_of(base0 + k * CH, 8)
        pltpu.make_async_copy(dst_h.at[pl.ds(b, CH)], idx_v.at[p],
                              ld_sem.at[p]).wait()
        pltpu.async_copy(ones_v, deg_sh.at[idx_v.at[p]], sc_sem.at[p],
                         add=True)

        @pl.when(k + 1 < N_CHUNKS)
        def _prefetch():
            @pl.when(k >= 1)
            def _drain():
                pltpu.make_async_copy(ones_v, deg_sh.at[idx_v.at[q]],
                                      sc_sem.at[q]).wait()
            bn = pl.multiple_of(base0 + (k + 1) * CH, 8)
            pltpu.async_copy(dst_h.at[pl.ds(bn, CH)], idx_v.at[q],
                             ld_sem.at[q])

        return carry

    lax.fori_loop(0, N_CHUNKS, chunk, 0)
    pltpu.make_async_copy(ones_v, deg_sh.at[idx_v.at[(N_CHUNKS - 2) % 2]],
                          sc_sem.at[(N_CHUNKS - 2) % 2]).wait()
    pltpu.make_async_copy(ones_v, deg_sh.at[idx_v.at[(N_CHUNKS - 1) % 2]],
                          sc_sem.at[(N_CHUNKS - 1) % 2]).wait()
    plsc.subcore_barrier()
    o0 = pl.multiple_of(c * N_PAD + r0, 8)
    pltpu.sync_copy(deg_sh.at[pl.ds(r0, N_SL)], zb1)
    pltpu.sync_copy(zb1, degp_h.at[pl.ds(o0, N_SL)])


_deg_call = pl.kernel(
    _deg_body,
    out_type=pltpu.MemorySpace.HBM((NC * N_PAD,), jnp.float32),
    mesh=_mesh,
    scratch_types=[
        pltpu.VMEM_SHARED((N_PAD,), jnp.float32),
        pltpu.VMEM((2, CH), jnp.int32),
        pltpu.VMEM((CH,), jnp.float32),
        pltpu.VMEM((N_SL,), jnp.float32),
        pltpu.SemaphoreType.DMA((2,)),
        pltpu.SemaphoreType.DMA((2,)),
    ],
    compiler_params=_sc_params,
)


def _msg_body(src_h, dst_h, z_h, zero4_h, Ap_h, z_sh, A_sh, sidx, didx,
              zrows, zb4, ls_sem, ld_sem, g_sem, sc_sem):
    c = lax.axis_index("c")
    s = lax.axis_index("s")
    r0 = pl.multiple_of(s * N_SL, 8)
    for j in range(8):
        rj = r0 + j * N_ST
        pltpu.sync_copy(z_h.at[pl.ds(rj, N_ST), :], zb4)
        pltpu.sync_copy(zb4, z_sh.at[pl.ds(rj, N_ST), :])
        pltpu.sync_copy(zero4_h.at[pl.ds(rj, N_ST), :], zb4)
        pltpu.sync_copy(zb4, A_sh.at[pl.ds(rj, N_ST), :])
    plsc.subcore_barrier()
    base0 = c * E_CORE + s * E_TILE
    b0 = pl.multiple_of(base0, 8)
    pltpu.async_copy(src_h.at[pl.ds(b0, CHM)], sidx.at[0], ls_sem.at[0])
    pltpu.async_copy(dst_h.at[pl.ds(b0, CHM)], didx.at[0], ld_sem.at[0])

    def chunk(k, carry):
        p = lax.rem(k, 2)
        q = 1 - p
        b = pl.multiple_of(base0 + k * CHM, 8)
        pltpu.make_async_copy(src_h.at[pl.ds(b, CHM)], sidx.at[p],
                              ls_sem.at[p]).wait()
        pltpu.make_async_copy(dst_h.at[pl.ds(b, CHM)], didx.at[p],
                              ld_sem.at[p]).wait()
        pltpu.async_copy(z_sh.at[sidx.at[p]], zrows.at[p], g_sem).wait()
        pltpu.async_copy(zrows.at[p], A_sh.at[didx.at[p]], sc_sem.at[p],
                         add=True)

        @pl.when(k + 1 < N_CHUNKS_M)
        def _prefetch():
            @pl.when(k >= 1)
            def _drain():
                pltpu.make_async_copy(zrows.at[q], A_sh.at[didx.at[q]],
                                      sc_sem.at[q]).wait()
            bn = pl.multiple_of(base0 + (k + 1) * CHM, 8)
            pltpu.async_copy(src_h.at[pl.ds(bn, CHM)], sidx.at[q],
                             ls_sem.at[q])
            pltpu.async_copy(dst_h.at[pl.ds(bn, CHM)], didx.at[q],
                             ld_sem.at[q])

        return carry

    lax.fori_loop(0, N_CHUNKS_M, chunk, 0)
    pltpu.make_async_copy(zrows.at[(N_CHUNKS_M - 2) % 2],
                          A_sh.at[didx.at[(N_CHUNKS_M - 2) % 2]],
                          sc_sem.at[(N_CHUNKS_M - 2) % 2]).wait()
    pltpu.make_async_copy(zrows.at[(N_CHUNKS_M - 1) % 2],
                          A_sh.at[didx.at[(N_CHUNKS_M - 1) % 2]],
                          sc_sem.at[(N_CHUNKS_M - 1) % 2]).wait()
    plsc.subcore_barrier()
    for j in range(8):
        rj = r0 + j * N_ST
        pltpu.sync_copy(A_sh.at[pl.ds(rj, N_ST), :], zb4)
        pltpu.sync_copy(zb4, Ap_h.at[c, pl.ds(rj, N_ST), :])


_msg_call = pl.kernel(
    _msg_body,
    out_type=pltpu.MemorySpace.HBM((NC, N_PAD, RW), jnp.float32),
    mesh=_mesh,
    scratch_types=[
        pltpu.VMEM_SHARED((N_PAD, RW), jnp.float32),
        pltpu.VMEM_SHARED((N_PAD, RW), jnp.float32),
        pltpu.VMEM((2, CHM), jnp.int32),
        pltpu.VMEM((2, CHM), jnp.int32),
        pltpu.VMEM((2, CHM, RW), jnp.float32),
        pltpu.VMEM((N_ST, RW), jnp.float32),
        pltpu.SemaphoreType.DMA((2,)),
        pltpu.SemaphoreType.DMA((2,)),
        pltpu.SemaphoreType.DMA,
        pltpu.SemaphoreType.DMA((2,)),
    ],
    compiler_params=_sc_params,
)


def _t_body(src_h, dst_h, dinv_h, tp_h, dinv_sh, t_sh, sidx, didx, dbuf,
            zb1, ls_sem, ld_sem, g_sem, sc_sem):
    c = lax.axis_index("c")
    s = lax.axis_index("s")
    r0 = pl.multiple_of(s * N_SL, 8)
    pltpu.sync_copy(dinv_h.at[pl.ds(r0, N_SL)], zb1)
    pltpu.sync_copy(zb1, dinv_sh.at[pl.ds(r0, N_SL)])
    _zero_vmem(zb1, N_SL)
    pltpu.sync_copy(zb1, t_sh.at[pl.ds(r0, N_SL)])
    plsc.subcore_barrier()
    base0 = c * E_CORE + s * E_TILE
    b0 = pl.multiple_of(base0, 8)
    pltpu.async_copy(src_h.at[pl.ds(b0, CH)], sidx.at[0], ls_sem.at[0])
    pltpu.async_copy(dst_h.at[pl.ds(b0, CH)], didx.at[0], ld_sem.at[0])

    def chunk(k, carry):
        p = lax.rem(k, 2)
        q = 1 - p
        b = pl.multiple_of(base0 + k * CH, 8)
        pltpu.make_async_copy(src_h.at[pl.ds(b, CH)], sidx.at[p],
                              ls_sem.at[p]).wait()
        pltpu.make_async_copy(dst_h.at[pl.ds(b, CH)], didx.at[p],
                              ld_sem.at[p]).wait()
        pltpu.async_copy(dinv_sh.at[didx.at[p]], dbuf.at[p], g_sem).wait()
        pltpu.async_copy(dbuf.at[p], t_sh.at[sidx.at[p]], sc_sem.at[p],
                         add=True)

        @pl.when(k + 1 < N_CHUNKS)
        def _prefetch():
            @pl.when(k >= 1)
            def _drain():
                pltpu.make_async_copy(dbuf.at[q], t_sh.at[sidx.at[q]],
                                      sc_sem.at[q]).wait()
            bn = pl.multiple_of(base0 + (k + 1) * CH, 8)
            pltpu.async_copy(src_h.at[pl.ds(bn, CH)], sidx.at[q],
                             ls_sem.at[q])
            pltpu.async_copy(dst_h.at[pl.ds(bn, CH)], didx.at[q],
                             ld_sem.at[q])

        return carry

    lax.fori_loop(0, N_CHUNKS, chunk, 0)
    pltpu.make_async_copy(dbuf.at[(N_CHUNKS - 2) % 2],
                          t_sh.at[sidx.at[(N_CHUNKS - 2) % 2]],
                          sc_sem.at[(N_CHUNKS - 2) % 2]).wait()
    pltpu.make_async_copy(dbuf.at[(N_CHUNKS - 1) % 2],
                          t_sh.at[sidx.at[(N_CHUNKS - 1) % 2]],
                          sc_sem.at[(N_CHUNKS - 1) % 2]).wait()
    plsc.subcore_barrier()
    o0 = pl.multiple_of(c * N_PAD + r0, 8)
    pltpu.sync_copy(t_sh.at[pl.ds(r0, N_SL)], zb1)
    pltpu.sync_copy(zb1, tp_h.at[pl.ds(o0, N_SL)])


_t_call = pl.kernel(
    _t_body,
    out_type=pltpu.MemorySpace.HBM((NC * N_PAD,), jnp.float32),
    mesh=_mesh,
    scratch_types=[
        pltpu.VMEM_SHARED((N_PAD,), jnp.float32),
        pltpu.VMEM_SHARED((N_PAD,), jnp.float32),
        pltpu.VMEM((2, CH), jnp.int32),
        pltpu.VMEM((2, CH), jnp.int32),
        pltpu.VMEM((2, CH), jnp.float32),
        pltpu.VMEM((N_SL,), jnp.float32),
        pltpu.SemaphoreType.DMA((2,)),
        pltpu.SemaphoreType.DMA((2,)),
        pltpu.SemaphoreType.DMA,
        pltpu.SemaphoreType.DMA((2,)),
    ],
    compiler_params=_sc_params,
)


_HI = lax.Precision.HIGHEST


def _prep_body(degp_ref, xv_ref, e8_ref, dinv_ref, z_ref):
    dp = degp_ref[...]
    deg = dp[0] + dp[1] + 1.0
    dinvR = lax.rsqrt(deg)
    dinv_ref[...] = dinvR
    dinv128 = jnp.dot(dinvR, e8_ref[...], precision=_HI,
                      preferred_element_type=jnp.float32)
    z_ref[...] = xv_ref[...] * dinv128


def _finish_body(Ap_ref, zv_ref, dinvR_ref, tp_ref, e8_ref, e16_ref,
                 fold_ref, wbig_ref, b1big_ref, W2_ref, b2_ref, Wl_ref,
                 bl_ref, out_ref):
    dinvR = dinvR_ref[...]
    S = Ap_ref[0] + Ap_ref[1] + zv_ref[...]
    Ssc = S * jnp.dot(dinvR, e8_ref[...], precision=_HI,
                      preferred_element_type=jnp.float32)
    H = jnp.maximum(
        jnp.dot(Ssc, wbig_ref[...], precision=_HI,
                preferred_element_type=jnp.float32)
        + b1big_ref[...][None, :], 0.0)
    tR = tp_ref[0] + tp_ref[1]
    dinvA = jnp.dot(dinvR, e16_ref[...], precision=_HI,
                    preferred_element_type=jnp.float32)
    tA = jnp.dot(tR, e16_ref[...], precision=_HI,
                 preferred_element_type=jnp.float32)
    wA = dinvA * (dinvA + tA)
    rowmask = (lax.broadcasted_iota(jnp.int32, (N_ROWS, 1), 0) < N_FULL)
    P = jnp.where(rowmask, H * wA, 0.0)
    c512 = jnp.sum(P, axis=0)
    c = jnp.dot(c512[None, :], fold_ref[...], precision=_HI,
                preferred_element_type=jnp.float32)
    G = jnp.dot(c, W2_ref[...].T, precision=_HI,
                preferred_element_type=jnp.float32) \
        + float(N_NODES) * b2_ref[...][None, :]
    L = jnp.dot(G, Wl_ref[...].T, precision=_HI,
                preferred_element_type=jnp.float32) \
        + bl_ref[...][None, :]
    m = jnp.max(L, axis=1, keepdims=True)
    out_ref[...] = L - m - jnp.log(jnp.sum(jnp.exp(L - m), axis=1,
                                           keepdims=True))


def kernel(x, edge_index, W1, b1, W2, b2, Wl, bl):
    f32 = jnp.float32
    src = edge_index[0].astype(jnp.int32)
    dst = edge_index[1].astype(jnp.int32)
    xv = jnp.pad(x.astype(f32), ((0, N_PAD - N_NODES), (0, RW - 3))) \
        .reshape(N_ROWS, 128)
    eyeN = jnp.eye(NPR, dtype=f32)
    e8 = jnp.kron(eyeN, jnp.ones((1, RW), f32))
    e16 = jnp.kron(eyeN, jnp.ones((1, 16), f32))
    fold = jnp.tile(jnp.eye(16, dtype=f32), (NPR, 1))
    W1p = jnp.pad(W1.astype(f32), ((0, 0), (0, RW - 3)))
    wbig = jnp.kron(eyeN, W1p.T)
    b1big = jnp.tile(b1.astype(f32), NPR)
    zeros4 = jnp.zeros((N_PAD, RW), f32)

    degp = _deg_call(dst).reshape(NC, N_ROWS, NPR)

    dinvR, zv = pl.pallas_call(
        _prep_body,
        out_shape=(
            jax.ShapeDtypeStruct((N_ROWS, NPR), f32),
            jax.ShapeDtypeStruct((N_ROWS, 128), f32),
        ),
    )(degp, xv, e8)

    hbm = pltpu.MemorySpace.HBM
    z_rows = pltpu.with_memory_space_constraint(zv.reshape(N_PAD, RW), hbm)
    dinv_flat = pltpu.with_memory_space_constraint(dinvR.reshape(N_PAD), hbm)
    zeros4 = pltpu.with_memory_space_constraint(zeros4, hbm)

    Ap = _msg_call(src, dst, z_rows, zeros4)
    tp = _t_call(src, dst, dinv_flat)

    Ap_v = Ap.reshape(NC, N_ROWS, 128)
    tp_v = tp.reshape(NC, N_ROWS, NPR)

    out = pl.pallas_call(
        _finish_body,
        out_shape=jax.ShapeDtypeStruct((1, 7), f32),
    )(Ap_v, zv, dinvR, tp_v, e8, e16, fold, wbig, b1big, W2, b2, Wl, bl)
    return out
